# Initial kernel scaffold; baseline (speedup 1.0000x reference)
import jax
import jax.numpy as jnp
from jax.experimental import pallas as pl


def kernel(user_ids, user_factors, item_factors):
    raise NotImplementedError("write your pallas kernel here")


# speedup vs baseline: 1010.6525x; 1010.6525x over previous
import functools

import jax
import jax.numpy as jnp
from jax import lax
from jax.experimental import pallas as pl
from jax.experimental.pallas import tpu as pltpu
from jax.experimental.pallas import tpu_sc as plsc


def _gather_rows_sc(table, idx):
    n_rows, d = table.shape
    b = idx.shape[0]
    info = plsc.get_sparse_core_info()
    nw = info.num_cores * info.num_subcores
    assert b % (8 * nw) == 0 and d % info.num_lanes == 0
    b_per_w = b // nw
    mesh = plsc.VectorSubcoreMesh(core_axis_name="c", subcore_axis_name="s")

    @functools.partial(
        pl.kernel,
        mesh=mesh,
        compiler_params=pltpu.CompilerParams(use_tc_tiling_on_sc=False),
        out_type=jax.ShapeDtypeStruct((b, d), jnp.float32),
        scratch_types=[
            pltpu.VMEM((b_per_w,), jnp.int32),
            pltpu.VMEM((b_per_w, d), jnp.float32),
            pltpu.SemaphoreType.DMA,
        ],
    )
    def gather_kernel(table_hbm, idx_hbm, out_hbm, idx_v, rows_v, sem):
        wid = lax.axis_index("s") * info.num_cores + lax.axis_index("c")
        base = wid * b_per_w
        pltpu.sync_copy(idx_hbm.at[pl.ds(base, b_per_w)], idx_v)
        pltpu.async_copy(table_hbm.at[idx_v], rows_v, sem).wait()
        pltpu.sync_copy(rows_v, out_hbm.at[pl.ds(base, b_per_w)])

    return gather_kernel(table, idx)


def _scores_matmul_kernel(uf_ref, items_ref, out_ref):
    out_ref[...] = lax.dot_general(
        uf_ref[...],
        items_ref[...],
        dimension_numbers=(((1,), (1,)), ((), ())),
        preferred_element_type=jnp.float32,
    )


def _scores_tc(uf, item_factors, block_items):
    b, d = uf.shape
    n_items = item_factors.shape[0]
    n_blocks = pl.cdiv(n_items, block_items)
    return pl.pallas_call(
        _scores_matmul_kernel,
        grid=(n_blocks,),
        in_specs=[
            pl.BlockSpec((b, d), lambda i: (0, 0)),
            pl.BlockSpec((block_items, d), lambda i: (i, 0)),
        ],
        out_specs=pl.BlockSpec((b, block_items), lambda i: (0, i)),
        out_shape=jax.ShapeDtypeStruct((b, n_items), jnp.float32),
    )(uf, item_factors)


def kernel(user_ids, user_factors, item_factors):
    uf = _gather_rows_sc(user_factors, user_ids.astype(jnp.int32))
    return _scores_tc(uf, item_factors, block_items=2048)


# speedup vs baseline: 1083.3793x; 1.0720x over previous
import functools

import jax
import jax.numpy as jnp
from jax import lax
from jax.experimental import pallas as pl
from jax.experimental.pallas import tpu as pltpu
from jax.experimental.pallas import tpu_sc as plsc


def _gather_rows_sc(table, idx):
    n_rows, d = table.shape
    b = idx.shape[0]
    info = plsc.get_sparse_core_info()
    nw = info.num_cores * info.num_subcores
    assert b % (8 * nw) == 0 and d % info.num_lanes == 0
    b_per_w = b // nw
    mesh = plsc.VectorSubcoreMesh(core_axis_name="c", subcore_axis_name="s")

    @functools.partial(
        pl.kernel,
        mesh=mesh,
        compiler_params=pltpu.CompilerParams(use_tc_tiling_on_sc=False),
        out_type=jax.ShapeDtypeStruct((b, d), jnp.float32),
        scratch_types=[
            pltpu.VMEM((b_per_w,), jnp.int32),
            pltpu.VMEM((b_per_w, d), jnp.float32),
            pltpu.SemaphoreType.DMA,
        ],
    )
    def gather_kernel(table_hbm, idx_hbm, out_hbm, idx_v, rows_v, sem):
        wid = lax.axis_index("s") * info.num_cores + lax.axis_index("c")
        base = wid * b_per_w
        pltpu.sync_copy(idx_hbm.at[pl.ds(base, b_per_w)], idx_v)
        pltpu.async_copy(table_hbm.at[idx_v], rows_v, sem).wait()
        pltpu.sync_copy(rows_v, out_hbm.at[pl.ds(base, b_per_w)])

    return gather_kernel(table, idx)


def _scores_matmul_kernel(uf_ref, items_t_ref, out_ref):
    out_ref[...] = lax.dot_general(
        uf_ref[...],
        items_t_ref[...],
        dimension_numbers=(((1,), (0,)), ((), ())),
        preferred_element_type=jnp.float32,
    )


def _scores_tc(uf, items_t, block_rows):
    b, d = uf.shape
    n_items = items_t.shape[1]
    n_blocks = pl.cdiv(b, block_rows)
    return pl.pallas_call(
        _scores_matmul_kernel,
        grid=(n_blocks,),
        in_specs=[
            pl.BlockSpec((block_rows, d), lambda i: (i, 0)),
            pl.BlockSpec((d, n_items), lambda i: (0, 0)),
        ],
        out_specs=pl.BlockSpec((block_rows, n_items), lambda i: (i, 0)),
        out_shape=jax.ShapeDtypeStruct((b, n_items), jnp.float32),
    )(uf, items_t)


def kernel(user_ids, user_factors, item_factors):
    uf = _gather_rows_sc(user_factors, user_ids.astype(jnp.int32))
    return _scores_tc(uf, item_factors.T, block_rows=32)


# speedup vs baseline: 1084.2702x; 1.0008x over previous
import functools

import jax
import jax.numpy as jnp
from jax import lax
from jax.experimental import pallas as pl
from jax.experimental.pallas import tpu as pltpu
from jax.experimental.pallas import tpu_sc as plsc


def _gather_rows_sc(table, idx):
    n_rows, d = table.shape
    b = idx.shape[0]
    info = plsc.get_sparse_core_info()
    nw = info.num_cores * info.num_subcores
    assert b % (8 * nw) == 0 and d % info.num_lanes == 0
    b_per_w = b // nw
    mesh = plsc.VectorSubcoreMesh(core_axis_name="c", subcore_axis_name="s")

    @functools.partial(
        pl.kernel,
        mesh=mesh,
        compiler_params=pltpu.CompilerParams(use_tc_tiling_on_sc=False),
        out_type=jax.ShapeDtypeStruct((b, d), jnp.float32),
        scratch_types=[
            pltpu.VMEM((b_per_w,), jnp.int32),
            pltpu.VMEM((b_per_w, d), jnp.float32),
            pltpu.SemaphoreType.DMA,
        ],
    )
    def gather_kernel(table_hbm, idx_hbm, out_hbm, idx_v, rows_v, sem):
        wid = lax.axis_index("s") * info.num_cores + lax.axis_index("c")
        base = wid * b_per_w
        pltpu.sync_copy(idx_hbm.at[pl.ds(base, b_per_w)], idx_v)
        pltpu.async_copy(table_hbm.at[idx_v], rows_v, sem).wait()
        pltpu.sync_copy(rows_v, out_hbm.at[pl.ds(base, b_per_w)])

    return gather_kernel(table, idx)


def _scores_matmul_kernel(uf_ref, items_t_ref, out_ref):
    out_ref[...] = lax.dot_general(
        uf_ref[...],
        items_t_ref[...],
        dimension_numbers=(((1,), (0,)), ((), ())),
        preferred_element_type=jnp.float32,
    )


def _scores_tc(uf, items_t, block_rows):
    b, d = uf.shape
    n_items = items_t.shape[1]
    n_blocks = pl.cdiv(b, block_rows)
    return pl.pallas_call(
        _scores_matmul_kernel,
        grid=(n_blocks,),
        in_specs=[
            pl.BlockSpec((block_rows, d), lambda i: (i, 0)),
            pl.BlockSpec((d, n_items), lambda i: (0, 0)),
        ],
        out_specs=pl.BlockSpec((block_rows, n_items), lambda i: (i, 0)),
        out_shape=jax.ShapeDtypeStruct((b, n_items), jnp.float32),
    )(uf, items_t)


def kernel(user_ids, user_factors, item_factors):
    uf = _gather_rows_sc(user_factors, user_ids.astype(jnp.int32))
    return _scores_tc(uf, item_factors.T, block_rows=64)
